# baseline (speedup 1.0000x reference)
import functools

import jax
import jax.numpy as jnp
from jax import lax
from jax.experimental import pallas as pl
from jax.experimental.pallas import tpu as pltpu
from jax.experimental.pallas import tpu_sc as plsc

LANES = 16
NUM_CORES = 2
NUM_SUBCORES = 16
NW = NUM_CORES * NUM_SUBCORES

B, T, V, D = 16, 2048, 100000, 64
TB = 128
NBLK = V // TB
MAIN_BLKS = 780
SLOTS = 25
TAIL0 = MAIN_BLKS * TB


@functools.lru_cache(maxsize=None)
def _build_relayout():
    mesh = plsc.VectorSubcoreMesh(core_axis_name="c", subcore_axis_name="s")

    @functools.partial(
        pl.kernel,
        mesh=mesh,
        compiler_params=pltpu.CompilerParams(needs_layout_passes=False),
        out_type=jax.ShapeDtypeStruct((V // 2, 2 * D), jnp.float32),
        scratch_types=[
            pltpu.VMEM((D, TB), jnp.float32),
            pltpu.VMEM((TB // 2, 2 * D), jnp.float32),
        ],
    )
    def k1(tokT_hbm, tail_hbm, tab_hbm, stg_v, tr_v):
        w = lax.axis_index("s") * NUM_CORES + lax.axis_index("c")
        iota16 = lax.broadcasted_iota(jnp.int32, (LANES,), 0)

        def transpose_block():
            def vb_body(vb, carry):
                tvec = vb * LANES + iota16
                rowi = tvec >> 1
                colb = (tvec & 1) * D
                base = vb * LANES

                def dd_body(dq, c2):
                    for u in range(8):
                        dd = dq * 8 + u
                        plsc.store_scatter(tr_v, [rowi, colb + dd],
                                           stg_v[dd, pl.ds(base, LANES)])
                    return c2

                lax.fori_loop(0, D // 8, dd_body, 0)
                return carry

            lax.fori_loop(0, TB // LANES, vb_body, 0)

        for j in range(SLOTS):
            g = w * SLOTS + j

            @pl.when(g < MAIN_BLKS)
            def _():
                pltpu.sync_copy(tokT_hbm.at[:, pl.ds(g * TB, TB)], stg_v)
                transpose_block()
                pltpu.sync_copy(tr_v, tab_hbm.at[pl.ds(g * (TB // 2),
                                                       TB // 2)])

        @pl.when(w == NW - 1)
        def _():
            pltpu.sync_copy(tail_hbm.at[:, pl.ds(0, TB)], stg_v)
            transpose_block()
            pltpu.sync_copy(tr_v, tab_hbm.at[pl.ds(TAIL0 // 2, TB // 2)])
            pltpu.sync_copy(tail_hbm.at[:, pl.ds(TB, TB)], stg_v)
            transpose_block()
            pltpu.sync_copy(tr_v.at[pl.ds(0, (V - TAIL0 - TB) // 2)],
                            tab_hbm.at[pl.ds((TAIL0 + TB) // 2,
                                             (V - TAIL0 - TB) // 2)])

    return k1


@functools.lru_cache(maxsize=None)
def _build_gather():
    mesh = plsc.VectorSubcoreMesh(core_axis_name="c", subcore_axis_name="s")

    @functools.partial(
        pl.kernel,
        mesh=mesh,
        compiler_params=pltpu.CompilerParams(needs_layout_passes=False),
        out_type=jax.ShapeDtypeStruct((B, D // LANES, T // TB, LANES, TB),
                                      jnp.float32),
        scratch_types=[
            pltpu.VMEM((D, TB), jnp.float32),
            pltpu.VMEM((2, TB), jnp.int32),
            pltpu.VMEM((2, TB), jnp.int32),
            pltpu.VMEM((2, TB, 2 * D), jnp.float32),
            pltpu.VMEM((D // LANES, LANES, TB), jnp.float32),
            pltpu.SemaphoreType.DMA,
            pltpu.SemaphoreType.DMA,
        ],
    )
    def k2(x_hbm, tab_hbm, posT_hbm, out_hbm,
           pos_v, xb_v, idx_v, grows_v, ltile_v, sem0, sem1):
        c = lax.axis_index("c")
        s = lax.axis_index("s")
        iota16 = lax.broadcasted_iota(jnp.int32, (LANES,), 0)
        pltpu.sync_copy(posT_hbm.at[:, pl.ds(TB * s, TB)], pos_v)

        sems = [sem0, sem1]
        handles = [None, None]

        def start_unit(i):
            buf = i % 2
            b = NUM_CORES * 4 * c + i
            pltpu.sync_copy(x_hbm.at[pl.ds(b * T + TB * s, TB)],
                            xb_v.at[buf])
            for grp in range(TB // LANES):
                sl = pl.ds(grp * LANES, LANES)
                idx_v[buf, sl] = xb_v[buf, sl] >> 1
            handles[buf] = pltpu.async_copy(
                tab_hbm.at[idx_v.at[buf]], grows_v.at[buf], sems[buf])

        start_unit(0)
        for i in range(B // NUM_CORES):
            buf = i % 2
            b = NUM_CORES * 4 * c + i
            if i + 1 < B // NUM_CORES:
                start_unit(i + 1)
            handles[buf].wait()
            grows = grows_v.at[buf]

            def j_body(j, carry):
                base = j * LANES
                tvec = base + iota16
                colb = (xb_v[buf, pl.ds(base, LANES)] & 1) * D
                sl = pl.ds(base, LANES)

                def dd_body(dq, c2):
                    for u in range(8):
                        dd = dq * 8 + u
                        val = plsc.load_gather(grows, [tvec, colb + dd])
                        ltile_v[dd // LANES, dd % LANES, sl] = (
                            val + pos_v[dd, sl])
                    return c2

                lax.fori_loop(0, D // 8, dd_body, 0)
                return carry

            lax.fori_loop(0, TB // LANES, j_body, 0)

            pltpu.sync_copy(ltile_v, out_hbm.at[b, :, s])

    return k2


def kernel(x, tok_emb, pos_emb):
    xf = x.astype(jnp.int32).reshape(B * T)
    tail = jnp.pad(tok_emb[TAIL0:].T, ((0, 0), (0, 2 * TB - (V - TAIL0))))
    tab = _build_relayout()(tok_emb.T, tail)
    out5 = _build_gather()(xf, tab, pos_emb.T)
    return out5.transpose(0, 2, 4, 1, 3).reshape(B, T, D)


# speedup vs baseline: 2.3401x; 2.3401x over previous
import functools

import jax
import jax.numpy as jnp
from jax import lax
from jax.experimental import pallas as pl
from jax.experimental.pallas import tpu as pltpu
from jax.experimental.pallas import tpu_sc as plsc

D_MODEL = 64
LANES = 16
NUM_CORES = 2
NUM_SUBCORES = 16
NUM_WORKERS = NUM_CORES * NUM_SUBCORES
NBUF = 3


@functools.lru_cache(maxsize=None)
def _build(B: int, T: int, V: int, D: int):
    assert T % NUM_WORKERS == 0 and D % LANES == 0
    CH = T // NUM_WORKERS
    assert CH % 8 == 0 and CH <= 128
    mesh = plsc.VectorSubcoreMesh(core_axis_name="c", subcore_axis_name="s")

    @functools.partial(
        pl.kernel,
        mesh=mesh,
        compiler_params=pltpu.CompilerParams(use_tc_tiling_on_sc=False),
        out_type=jax.ShapeDtypeStruct((B, T, D), jnp.float32),
        scratch_types=[
            pltpu.VMEM((B, CH), jnp.int32),
            pltpu.VMEM((CH, D), jnp.float32),
            pltpu.VMEM((NBUF, CH, D), jnp.float32),
            pltpu.SemaphoreType.DMA,
            pltpu.SemaphoreType.DMA,
            pltpu.SemaphoreType.DMA,
            pltpu.SemaphoreType.DMA,
            pltpu.SemaphoreType.DMA,
            pltpu.SemaphoreType.DMA,
        ],
    )
    def k(x_hbm, tok_hbm, pos_hbm, out_hbm, idx_v, pos_v, rows_v,
          sg0, sg1, sg2, so0, so1, so2):
        w = lax.axis_index("s") * NUM_CORES + lax.axis_index("c")
        t0 = w * CH
        pltpu.sync_copy(pos_hbm.at[pl.ds(t0, CH)], pos_v)
        pltpu.sync_copy(x_hbm.at[:, pl.ds(t0, CH)], idx_v)

        gsems = [sg0, sg1, sg2]
        osems = [so0, so1, so2]
        ghandles = [None] * B
        ohandles = [None] * B

        def start_gather(b):
            buf = b % NBUF
            ghandles[b] = pltpu.async_copy(
                tok_hbm.at[idx_v.at[b]], rows_v.at[buf], gsems[buf])

        def start_out(b):
            buf = b % NBUF
            ohandles[b] = pltpu.async_copy(
                rows_v.at[buf], out_hbm.at[b, pl.ds(t0, CH)], osems[buf])

        start_gather(0)
        start_gather(1)
        for b in range(B):
            buf = b % NBUF
            ghandles[b].wait()
            rows = rows_v.at[buf]

            def body(r, carry):
                for kk in range(D // LANES):
                    sl = pl.ds(kk * LANES, LANES)
                    rows[r, sl] = rows[r, sl] + pos_v[r, sl]
                return carry

            lax.fori_loop(0, CH, body, 0)
            start_out(b)
            if b + 2 < B:
                if b >= 1:
                    ohandles[b - 1].wait()
                start_gather(b + 2)
        ohandles[B - 3].wait()
        ohandles[B - 2].wait()
        ohandles[B - 1].wait()

    return k


def kernel(x, tok_emb, pos_emb):
    B, T = x.shape
    V, D = tok_emb.shape
    k = _build(B, T, V, D)
    return k(x.astype(jnp.int32), tok_emb, pos_emb)


# speedup vs baseline: 2.4186x; 1.0336x over previous
import functools

import jax
import jax.numpy as jnp
from jax import lax
from jax.experimental import pallas as pl
from jax.experimental.pallas import tpu as pltpu
from jax.experimental.pallas import tpu_sc as plsc

D_MODEL = 64
LANES = 16
NUM_CORES = 2
NUM_SUBCORES = 16
NUM_WORKERS = NUM_CORES * NUM_SUBCORES
NBUF = 3


@functools.lru_cache(maxsize=None)
def _build(B: int, T: int, V: int, D: int):
    assert T % NUM_WORKERS == 0 and D % LANES == 0
    CH = T // NUM_WORKERS
    assert CH % 8 == 0 and CH <= 128
    mesh = plsc.VectorSubcoreMesh(core_axis_name="c", subcore_axis_name="s")

    @functools.partial(
        pl.kernel,
        mesh=mesh,
        compiler_params=pltpu.CompilerParams(use_tc_tiling_on_sc=False),
        out_type=jax.ShapeDtypeStruct((B, T, D), jnp.float32),
        scratch_types=[
            pltpu.VMEM((B, CH), jnp.int32),
            pltpu.VMEM((B // 2, 2 * CH), jnp.int32),
            pltpu.VMEM((CH, D), jnp.float32),
            pltpu.VMEM((NBUF, 2 * CH, D), jnp.float32),
            pltpu.SemaphoreType.DMA,
            pltpu.SemaphoreType.DMA,
            pltpu.SemaphoreType.DMA,
            pltpu.SemaphoreType.DMA,
            pltpu.SemaphoreType.DMA,
            pltpu.SemaphoreType.DMA,
        ],
    )
    def k(x_hbm, tok_hbm, pos_hbm, out_hbm, idx_v, idx2_v, pos_v, rows_v,
          sg0, sg1, sg2, so0, so1, so2):
        w = lax.axis_index("s") * NUM_CORES + lax.axis_index("c")
        t0 = w * CH
        pltpu.sync_copy(pos_hbm.at[pl.ds(t0, CH)], pos_v)
        pltpu.sync_copy(x_hbm.at[:, pl.ds(t0, CH)], idx_v)
        for bb in range(B):
            for kk in range(CH // LANES):
                idx2_v[bb // 2,
                       pl.ds((bb % 2) * CH + kk * LANES, LANES)] = (
                    idx_v[bb, pl.ds(kk * LANES, LANES)])

        gsems = [sg0, sg1, sg2]
        osems = [so0, so1, so2]
        NCH = B // 2
        ghandles = [None] * NCH
        ohandles = [None] * NCH

        def start_gather(cidx):
            buf = cidx % NBUF
            ghandles[cidx] = pltpu.async_copy(
                tok_hbm.at[idx2_v.at[cidx]], rows_v.at[buf], gsems[buf])

        def start_out(cidx):
            buf = cidx % NBUF
            h0 = pltpu.async_copy(
                rows_v.at[buf, pl.ds(0, CH)],
                out_hbm.at[2 * cidx, pl.ds(t0, CH)], osems[buf])
            h1 = pltpu.async_copy(
                rows_v.at[buf, pl.ds(CH, CH)],
                out_hbm.at[2 * cidx + 1, pl.ds(t0, CH)], osems[buf])
            ohandles[cidx] = (h0, h1)

        start_gather(0)
        start_gather(1)
        for cidx in range(NCH):
            buf = cidx % NBUF
            ghandles[cidx].wait()
            rows = rows_v.at[buf]

            def body(r, carry):
                for half in range(2):
                    for kk in range(D // LANES):
                        sl = pl.ds(kk * LANES, LANES)
                        rr = half * CH + r
                        rows[rr, sl] = rows[rr, sl] + pos_v[r, sl]
                return carry

            lax.fori_loop(0, CH, body, 0)
            start_out(cidx)
            if cidx + 2 < NCH:
                if cidx >= 1:
                    ohandles[cidx - 1][0].wait()
                    ohandles[cidx - 1][1].wait()
                start_gather(cidx + 2)
        for t in (NCH - 3, NCH - 2, NCH - 1):
            ohandles[t][0].wait()
            ohandles[t][1].wait()

    return k


def kernel(x, tok_emb, pos_emb):
    B, T = x.shape
    V, D = tok_emb.shape
    k = _build(B, T, V, D)
    return k(x.astype(jnp.int32), tok_emb, pos_emb)


# speedup vs baseline: 2.4459x; 1.0113x over previous
import functools

import jax
import jax.numpy as jnp
from jax import lax
from jax.experimental import pallas as pl
from jax.experimental.pallas import tpu as pltpu
from jax.experimental.pallas import tpu_sc as plsc

D_MODEL = 64
LANES = 16
NUM_CORES = 2
NUM_SUBCORES = 16
NUM_WORKERS = NUM_CORES * NUM_SUBCORES
NBUF = 3


@functools.lru_cache(maxsize=None)
def _build(B: int, T: int, V: int, D: int):
    assert T % NUM_WORKERS == 0 and D % LANES == 0
    CH = T // NUM_WORKERS
    assert CH % 8 == 0 and CH <= 128
    mesh = plsc.VectorSubcoreMesh(core_axis_name="c", subcore_axis_name="s")

    @functools.partial(
        pl.kernel,
        mesh=mesh,
        compiler_params=pltpu.CompilerParams(use_tc_tiling_on_sc=False),
        out_type=jax.ShapeDtypeStruct((B, T, D), jnp.float32),
        scratch_types=[
            pltpu.VMEM((B, CH), jnp.int32),
            pltpu.VMEM((B // 2, 2 * CH), jnp.int32),
            pltpu.VMEM((CH, D), jnp.float32),
            pltpu.VMEM((8, 2 * CH, D), jnp.float32),
        ] + [pltpu.SemaphoreType.DMA] * 10,
    )
    def k(x_hbm, tok_hbm, pos_hbm, out_hbm, idx_v, idx2_v, pos_v, rows_v,
          *sems):
        w = lax.axis_index("s") * NUM_CORES + lax.axis_index("c")
        t0 = w * CH
        pltpu.sync_copy(pos_hbm.at[pl.ds(t0, CH)], pos_v)
        pltpu.sync_copy(x_hbm.at[:, pl.ds(t0, CH)], idx_v)
        for bb in range(B):
            for kk in range(CH // LANES):
                idx2_v[bb // 2,
                       pl.ds((bb % 2) * CH + kk * LANES, LANES)] = (
                    idx_v[bb, pl.ds(kk * LANES, LANES)])

        NCH = B // 2
        gsems = sems[:8]
        osems = sems[8:]
        ghandles = [None] * NCH
        ohandles = [None] * NCH

        for cidx in range(NCH):
            ghandles[cidx] = pltpu.async_copy(
                tok_hbm.at[idx2_v.at[cidx]], rows_v.at[cidx], gsems[cidx])

        for cidx in range(NCH):
            ghandles[cidx].wait()
            rows = rows_v.at[cidx]

            def body(r, carry):
                for half in range(2):
                    for kk in range(D // LANES):
                        sl = pl.ds(kk * LANES, LANES)
                        rr = half * CH + r
                        rows[rr, sl] = rows[rr, sl] + pos_v[r, sl]
                return carry

            lax.fori_loop(0, CH, body, 0)
            h0 = pltpu.async_copy(
                rows_v.at[cidx, pl.ds(0, CH)],
                out_hbm.at[2 * cidx, pl.ds(t0, CH)], osems[cidx % 2])
            h1 = pltpu.async_copy(
                rows_v.at[cidx, pl.ds(CH, CH)],
                out_hbm.at[2 * cidx + 1, pl.ds(t0, CH)], osems[cidx % 2])
            ohandles[cidx] = (h0, h1)
        for cidx in range(NCH):
            ohandles[cidx][0].wait()
            ohandles[cidx][1].wait()

    return k


def kernel(x, tok_emb, pos_emb):
    B, T = x.shape
    V, D = tok_emb.shape
    k = _build(B, T, V, D)
    return k(x.astype(jnp.int32), tok_emb, pos_emb)


# speedup vs baseline: 2.4513x; 1.0022x over previous
import functools

import jax
import jax.numpy as jnp
from jax import lax
from jax.experimental import pallas as pl
from jax.experimental.pallas import tpu as pltpu
from jax.experimental.pallas import tpu_sc as plsc

D_MODEL = 64
LANES = 16
NUM_CORES = 2
NUM_SUBCORES = 16
NUM_WORKERS = NUM_CORES * NUM_SUBCORES


@functools.lru_cache(maxsize=None)
def _build(B: int, T: int, V: int, D: int):
    assert T % NUM_WORKERS == 0 and D % LANES == 0
    CH = T // NUM_WORKERS
    assert CH % 8 == 0 and CH <= 128
    mesh = plsc.VectorSubcoreMesh(core_axis_name="c", subcore_axis_name="s")

    @functools.partial(
        pl.kernel,
        mesh=mesh,
        compiler_params=pltpu.CompilerParams(use_tc_tiling_on_sc=False),
        out_type=jax.ShapeDtypeStruct((B, T, D), jnp.float32),
        scratch_types=[
            pltpu.VMEM((B, CH), jnp.int32),
            pltpu.VMEM((B // 2, 2 * CH), jnp.int32),
            pltpu.VMEM((CH, D), jnp.float32),
            pltpu.VMEM((8, 2 * CH, D), jnp.float32),
        ] + [pltpu.SemaphoreType.DMA] * 10,
    )
    def k(x_hbm, tok_hbm, pos_hbm, out_hbm, idx_v, idx2_v, pos_v, rows_v,
          *sems):
        w = lax.axis_index("s") * NUM_CORES + lax.axis_index("c")
        t0 = w * CH
        pltpu.sync_copy(pos_hbm.at[pl.ds(t0, CH)], pos_v)
        pltpu.sync_copy(x_hbm.at[:, pl.ds(t0, CH)], idx_v)
        for bb in range(B):
            for kk in range(CH // LANES):
                idx2_v[bb // 2,
                       pl.ds((bb % 2) * CH + kk * LANES, LANES)] = (
                    idx_v[bb, pl.ds(kk * LANES, LANES)])

        NCH = B // 2
        gsems = sems[:8]
        osems = sems[8:]
        ghandles = [None] * NCH
        ohandles = [None] * NCH

        for cidx in range(NCH):
            ghandles[cidx] = pltpu.async_copy(
                tok_hbm.at[idx2_v.at[cidx]], rows_v.at[cidx], gsems[cidx])

        for cidx in range(NCH):
            ghandles[cidx].wait()
            rows = rows_v.at[cidx]

            def body(r, carry):
                for half in range(2):
                    for kk in range(D // LANES):
                        sl = pl.ds(kk * LANES, LANES)
                        rr = half * CH + r
                        rows[rr, sl] = rows[rr, sl] + pos_v[r, sl]
                return carry

            lax.fori_loop(0, CH, body, 0)
            h0 = pltpu.async_copy(
                rows_v.at[cidx, pl.ds(0, CH)],
                out_hbm.at[2 * cidx, pl.ds(t0, CH)], osems[cidx % 2])
            h1 = pltpu.async_copy(
                rows_v.at[cidx, pl.ds(CH, CH)],
                out_hbm.at[2 * cidx + 1, pl.ds(t0, CH)], osems[cidx % 2])
            ohandles[cidx] = (h0, h1)
        for cidx in range(NCH):
            ohandles[cidx][0].wait()
            ohandles[cidx][1].wait()

    return k


def kernel(x, tok_emb, pos_emb):
    B, T = x.shape
    V, D = tok_emb.shape
    k = _build(B, T, V, D)
    return k(x.astype(jnp.int32), tok_emb, pos_emb)
